# baseline (speedup 1.0000x reference)
import functools

import jax
import jax.numpy as jnp
from jax import lax
from jax.experimental import pallas as pl
from jax.experimental.pallas import tpu as pltpu
from jax.experimental.pallas import tpu_sc as plsc

_C = 128
_NW = 32
_SB = 8
_EPS = 1e-5


def _sc_message_pass(xk, src, dst, hidx, n_cols, gpt, e_groups):
    n, d = xk.shape
    rows_pad = n + 128
    fl_rows = (n // 16) // 8 * 8
    tail0 = fl_rows * 16
    hflat = rows_pad * n_cols
    nsb = gpt // _SB

    mesh = plsc.VectorSubcoreMesh(core_axis_name="c", subcore_axis_name="s")

    @functools.partial(
        pl.kernel,
        out_type=(
            jax.ShapeDtypeStruct((2, n, d), jnp.float32),
            jax.ShapeDtypeStruct((2, n * n_cols), jnp.float32),
        ),
        mesh=mesh,
        scratch_types=(
            pltpu.VMEM((2, _SB, _C), jnp.int32),
            pltpu.VMEM((2, _SB, _C), jnp.int32),
            pltpu.VMEM((2, _SB, _C), jnp.int32),
            pltpu.VMEM((2, _C, d), jnp.float32),
            pltpu.VMEM((_C,), jnp.float32),
            pltpu.VMEM((_C,), jnp.float32),
            pltpu.VMEM_SHARED((rows_pad, d), jnp.float32),
            pltpu.VMEM_SHARED((hflat,), jnp.float32),
            pltpu.SemaphoreType.DMA((2,)),
            pltpu.SemaphoreType.DMA((2,)),
            pltpu.SemaphoreType.DMA((2,)),
        ),
    )
    def run(x_hbm, src_hbm, dst_hbm, hidx_hbm, zr_hbm, zh_hbm, pos_hbm,
            neg_hbm, p_hbm, h_hbm,
            src_i, dst_i, hid_i, bufs_v, pos_v, neg_v, acc_sh, hist_sh,
            gsem, ssem, isem):
        c = lax.axis_index("c")
        s = lax.axis_index("s")
        w = (1 - c) * 16 + s
        rbase = w * gpt

        pltpu.sync_copy(pos_hbm, pos_v)
        pltpu.sync_copy(neg_hbm, neg_v)

        z0 = bufs_v.at[0]
        pltpu.sync_copy(zr_hbm, z0)
        zb = s * fl_rows
        nchunks = (fl_rows + _C - 1) // _C
        for k in range(nchunks):
            m = min(_C, fl_rows - k * _C)
            pltpu.sync_copy(z0.at[pl.ds(0, m)],
                            acc_sh.at[pl.ds(zb + k * _C, m)])

        @pl.when(s == 15)
        def _zero_tail():
            left = rows_pad - tail0
            off = tail0
            while left > 0:
                mm = min(_C, left)
                pltpu.sync_copy(z0.at[pl.ds(0, mm)],
                                acc_sh.at[pl.ds(off, mm)])
                off += mm
                left -= mm

        @pl.when(s == 0)
        def _zero_hist():
            pltpu.sync_copy(zh_hbm, hist_sh)

        plsc.subcore_barrier()

        pltpu.sync_copy(src_hbm.at[pl.ds(rbase, _SB)], src_i.at[0])
        pltpu.sync_copy(dst_hbm.at[pl.ds(rbase, _SB)], dst_i.at[0])
        pltpu.sync_copy(hidx_hbm.at[pl.ds(rbase, _SB)], hid_i.at[0])
        pltpu.async_copy(x_hbm.at[src_i.at[0, 0]], bufs_v.at[0], gsem.at[0])

        def body(g, carry):
            sb = g // _SB
            pos = g - sb * _SB
            slot = lax.rem(sb, 2)
            b = lax.rem(g, 2)

            @pl.when(jnp.logical_and(pos == 0, g + _SB < gpt))
            def _prefetch():
                nslot = 1 - slot
                nrow = rbase + (sb + 1) * _SB
                pltpu.async_copy(src_hbm.at[pl.ds(nrow, _SB)],
                                 src_i.at[nslot], isem.at[nslot])
                pltpu.async_copy(dst_hbm.at[pl.ds(nrow, _SB)],
                                 dst_i.at[nslot], isem.at[nslot])
                pltpu.async_copy(hidx_hbm.at[pl.ds(nrow, _SB)],
                                 hid_i.at[nslot], isem.at[nslot])

            pltpu.make_async_copy(x_hbm.at[src_i.at[slot, pos]],
                                  bufs_v.at[b], gsem.at[b]).wait()
            pltpu.async_copy(bufs_v.at[b], acc_sh.at[dst_i.at[slot, pos]],
                             ssem.at[b], add=True)
            gg = rbase + g

            @pl.when(gg < e_groups)
            def _pos_hist():
                pltpu.sync_copy(pos_v, hist_sh.at[hid_i.at[slot, pos]],
                                add=True)

            @pl.when(gg >= e_groups)
            def _neg_hist():
                pltpu.sync_copy(neg_v, hist_sh.at[hid_i.at[slot, pos]],
                                add=True)

            @pl.when(g + 1 < gpt)
            def _next_gather():
                bn = 1 - b

                @pl.when(g > 0)
                def _wait_prev_scatter():
                    pltpu.make_async_copy(zr_hbm, bufs_v.at[bn],
                                          ssem.at[bn]).wait()

                gn = g + 1
                sbn = gn // _SB
                posn = gn - sbn * _SB
                slotn = lax.rem(sbn, 2)

                @pl.when(posn == 0)
                def _wait_idx():
                    pltpu.make_async_copy(src_hbm.at[pl.ds(0, _SB)],
                                          src_i.at[slotn],
                                          isem.at[slotn]).wait()
                    pltpu.make_async_copy(dst_hbm.at[pl.ds(0, _SB)],
                                          dst_i.at[slotn],
                                          isem.at[slotn]).wait()
                    pltpu.make_async_copy(hidx_hbm.at[pl.ds(0, _SB)],
                                          hid_i.at[slotn],
                                          isem.at[slotn]).wait()

                pltpu.async_copy(x_hbm.at[src_i.at[slotn, posn]],
                                 bufs_v.at[bn], gsem.at[bn])

            return carry

        lax.fori_loop(0, gpt, body, 0)
        for b in range(2):
            pltpu.make_async_copy(zr_hbm, bufs_v.at[b], ssem.at[b]).wait()
        plsc.subcore_barrier()

        fb = s * fl_rows
        pltpu.sync_copy(acc_sh.at[pl.ds(fb, fl_rows)],
                        p_hbm.at[c, pl.ds(fb, fl_rows)])
        pltpu.sync_copy(hist_sh.at[pl.ds(fb * n_cols, fl_rows * n_cols)],
                        h_hbm.at[c, pl.ds(fb * n_cols, fl_rows * n_cols)])

        @pl.when(s == 15)
        def _flush_tail():
            pltpu.sync_copy(acc_sh.at[pl.ds(tail0, n - tail0)],
                            p_hbm.at[c, pl.ds(tail0, n - tail0)])
            pltpu.sync_copy(
                hist_sh.at[pl.ds(tail0 * n_cols, (n - tail0) * n_cols)],
                h_hbm.at[c, pl.ds(tail0 * n_cols, (n - tail0) * n_cols)])

    zr = jnp.zeros((_C, d), jnp.float32)
    zh = jnp.zeros((hflat,), jnp.float32)
    pos1 = jnp.ones((_C,), jnp.float32)
    neg1 = jnp.full((_C,), -1.0, jnp.float32)
    return run(xk, src, dst, hidx, zr, zh, pos1, neg1)


def _tc_prepass(x, kernels, bn_gamma, bn_beta):
    n, d = x.shape

    def body(x_ref, k_ref, g_ref, b_ref, o_ref):
        xv = x_ref[...]
        mean = jnp.mean(xv, axis=0)
        xc = xv - mean[None, :]
        var = jnp.mean(xc * xc, axis=0)
        sc = g_ref[0, :] / jnp.sqrt(var + _EPS)
        o_ref[...] = (xc * sc[None, :] + b_ref[0, :][None, :]) * k_ref[0, :]

    return pl.pallas_call(
        body,
        out_shape=jax.ShapeDtypeStruct((n, d), jnp.float32),
    )(x, kernels, bn_gamma, bn_beta)


def _tc_combine(xk, p, h, r, kernels, escal):
    n, d = xk.shape

    def body(x_ref, p_ref, h_ref, r_ref, k_ref, es_ref, o_ref):
        rk = r_ref[...] * k_ref[0, :][None, :]
        pv = p_ref[0] + p_ref[1]
        hv = h_ref[0] + h_ref[1]
        relpart = jnp.dot(hv, rk, preferred_element_type=jnp.float32,
                          precision=lax.Precision.HIGHEST)
        num = pv + relpart + x_ref[...]
        esv = es_ref[...]
        iot = lax.broadcasted_iota(jnp.int32, (n, 1), 0)
        du = jnp.ones((n, 1), jnp.float32)
        for i in range(3):
            ai = esv[2 * i:2 * i + 1, :]
            bi = esv[2 * i + 1:2 * i + 2, :]
            inc = (ai != bi).astype(jnp.float32)
            du = du + inc * ((iot == ai).astype(jnp.float32)
                             + (iot == bi).astype(jnp.float32))
        o_ref[...] = num / du

    return pl.pallas_call(
        body,
        out_shape=jax.ShapeDtypeStruct((n, d), jnp.float32),
    )(xk, p, h, r, kernels, escal)


def kernel(x, edges, rels, r, kernels, bn_gamma, bn_beta):
    n, d = x.shape
    e = edges.shape[1]
    nrel = r.shape[0]
    e0 = edges[0].astype(jnp.int32)
    e1 = edges[1].astype(jnp.int32)
    rl = rels.astype(jnp.int32)
    m = 2 * e
    chunk = _NW * _C * _SB
    mp = -(-m // chunk) * chunk
    gpt = mp // (_NW * _C)
    pad = mp - m
    dump = n + jnp.arange(pad, dtype=jnp.int32) % 128
    src = jnp.concatenate([e0, e1, jnp.zeros((pad,), jnp.int32)])
    dst = jnp.concatenate([e1, e0, dump])
    col = jnp.concatenate([rl, rl, jnp.zeros((pad,), jnp.int32)])
    hidx = dst * nrel + col
    xk = _tc_prepass(x, kernels, bn_gamma.reshape(1, d),
                     bn_beta.reshape(1, d))
    p, hf = _sc_message_pass(xk, src.reshape(-1, _C), dst.reshape(-1, _C),
                             hidx.reshape(-1, _C), nrel, gpt, e // _C)
    h = hf.reshape(2, n, nrel)
    escal = jnp.stack([e0[0], e0[2], rl[0], rl[2], e1[0], e1[2],
                       jnp.zeros((), jnp.int32), jnp.zeros((), jnp.int32)])
    return _tc_combine(xk, p, h, r, kernels, escal.reshape(8, 1))


# speedup vs baseline: 3.5392x; 3.5392x over previous
import functools

import jax
import jax.numpy as jnp
from jax import lax
from jax.experimental import pallas as pl
from jax.experimental.pallas import tpu as pltpu
from jax.experimental.pallas import tpu_sc as plsc

_C = 128
_NW = 32
_SB = 8
_EPS = 1e-5


def _sc_message_pass(xk, src, dst, hidx, n_cols, gpt, e_groups, rg):
    n, d = xk.shape
    rows_pad = n + 128
    fl_rows = (n // 16) // 8 * 8
    tail0 = fl_rows * 16
    hflat = rows_pad * n_cols
    nsb = gpt // _SB

    mesh = plsc.VectorSubcoreMesh(core_axis_name="c", subcore_axis_name="s")

    @functools.partial(
        pl.kernel,
        out_type=(
            jax.ShapeDtypeStruct((2, n, d), jnp.float32),
            jax.ShapeDtypeStruct((2, n * n_cols), jnp.float32),
        ),
        mesh=mesh,
        scratch_types=(
            pltpu.VMEM((2, _SB, _C), jnp.int32),
            pltpu.VMEM((2, _SB, _C), jnp.int32),
            pltpu.VMEM((2, _SB, _C), jnp.int32),
            pltpu.VMEM((2, _C, d), jnp.float32),
            pltpu.VMEM((_C,), jnp.float32),
            pltpu.VMEM((_C,), jnp.float32),
            pltpu.VMEM_SHARED((rows_pad, d), jnp.float32),
            pltpu.VMEM_SHARED((hflat,), jnp.float32),
            pltpu.SemaphoreType.DMA((2,)),
            pltpu.SemaphoreType.DMA((2,)),
            pltpu.SemaphoreType.DMA((2,)),
        ),
    )
    def run(x_hbm, src_hbm, dst_hbm, hidx_hbm, zr_hbm, zh_hbm, pos_hbm,
            neg_hbm, p_hbm, h_hbm,
            src_i, dst_i, hid_i, bufs_v, pos_v, neg_v, acc_sh, hist_sh,
            gsem, ssem, isem):
        c = lax.axis_index("c")
        s = lax.axis_index("s")
        w = c * 16 + s
        rbase = w * gpt

        pltpu.sync_copy(pos_hbm, pos_v)
        pltpu.sync_copy(neg_hbm, neg_v)

        z0 = bufs_v.at[0]
        pltpu.sync_copy(zr_hbm, z0)
        zb = s * fl_rows
        nchunks = (fl_rows + _C - 1) // _C
        for k in range(nchunks):
            m = min(_C, fl_rows - k * _C)
            pltpu.sync_copy(z0.at[pl.ds(0, m)],
                            acc_sh.at[pl.ds(zb + k * _C, m)])

        @pl.when(s == 15)
        def _zero_tail():
            left = rows_pad - tail0
            off = tail0
            while left > 0:
                mm = min(_C, left)
                pltpu.sync_copy(z0.at[pl.ds(0, mm)],
                                acc_sh.at[pl.ds(off, mm)])
                off += mm
                left -= mm

        @pl.when(s == 0)
        def _zero_hist():
            pltpu.sync_copy(zh_hbm, hist_sh)

        plsc.subcore_barrier()

        pltpu.sync_copy(src_hbm.at[pl.ds(rbase, _SB)], src_i.at[0])
        pltpu.sync_copy(dst_hbm.at[pl.ds(rbase, _SB)], dst_i.at[0])
        pltpu.sync_copy(hidx_hbm.at[pl.ds(rbase, _SB)], hid_i.at[0])
        pltpu.async_copy(x_hbm.at[src_i.at[0, 0]], bufs_v.at[0], gsem.at[0])

        def body(g, carry):
            sb = g // _SB
            pos = g - sb * _SB
            slot = lax.rem(sb, 2)
            b = lax.rem(g, 2)
            gg = rbase + g

            @pl.when(jnp.logical_and(pos == 0,
                                     jnp.logical_and(g + _SB < gpt,
                                                     gg + _SB < rg)))
            def _prefetch():
                nslot = 1 - slot
                nrow = rbase + (sb + 1) * _SB
                pltpu.async_copy(src_hbm.at[pl.ds(nrow, _SB)],
                                 src_i.at[nslot], isem.at[nslot])
                pltpu.async_copy(dst_hbm.at[pl.ds(nrow, _SB)],
                                 dst_i.at[nslot], isem.at[nslot])
                pltpu.async_copy(hidx_hbm.at[pl.ds(nrow, _SB)],
                                 hid_i.at[nslot], isem.at[nslot])

            @pl.when(gg < rg)
            def _process():
                pltpu.make_async_copy(x_hbm.at[src_i.at[slot, pos]],
                                      bufs_v.at[b], gsem.at[b]).wait()
                pltpu.async_copy(bufs_v.at[b], acc_sh.at[dst_i.at[slot, pos]],
                                 ssem.at[b], add=True)

                @pl.when(gg < e_groups)
                def _pos_hist():
                    pltpu.sync_copy(pos_v, hist_sh.at[hid_i.at[slot, pos]],
                                    add=True)

                @pl.when(gg >= e_groups)
                def _neg_hist():
                    pltpu.sync_copy(neg_v, hist_sh.at[hid_i.at[slot, pos]],
                                    add=True)

            @pl.when(jnp.logical_and(g + 1 < gpt, gg + 1 < rg))
            def _next_gather():
                bn = 1 - b

                @pl.when(g > 0)
                def _wait_prev_scatter():
                    pltpu.make_async_copy(zr_hbm, bufs_v.at[bn],
                                          ssem.at[bn]).wait()

                gn = g + 1
                sbn = gn // _SB
                posn = gn - sbn * _SB
                slotn = lax.rem(sbn, 2)

                @pl.when(posn == 0)
                def _wait_idx():
                    pltpu.make_async_copy(src_hbm.at[pl.ds(0, _SB)],
                                          src_i.at[slotn],
                                          isem.at[slotn]).wait()
                    pltpu.make_async_copy(dst_hbm.at[pl.ds(0, _SB)],
                                          dst_i.at[slotn],
                                          isem.at[slotn]).wait()
                    pltpu.make_async_copy(hidx_hbm.at[pl.ds(0, _SB)],
                                          hid_i.at[slotn],
                                          isem.at[slotn]).wait()

                pltpu.async_copy(x_hbm.at[src_i.at[slotn, posn]],
                                 bufs_v.at[bn], gsem.at[bn])

            return carry

        lax.fori_loop(0, gpt, body, 0)
        for b in range(2):
            pltpu.make_async_copy(zr_hbm, bufs_v.at[b], ssem.at[b]).wait()
        plsc.subcore_barrier()

        fb = s * fl_rows
        pltpu.sync_copy(acc_sh.at[pl.ds(fb, fl_rows)],
                        p_hbm.at[c, pl.ds(fb, fl_rows)])
        pltpu.sync_copy(hist_sh.at[pl.ds(fb * n_cols, fl_rows * n_cols)],
                        h_hbm.at[c, pl.ds(fb * n_cols, fl_rows * n_cols)])

        @pl.when(s == 15)
        def _flush_tail():
            pltpu.sync_copy(acc_sh.at[pl.ds(tail0, n - tail0)],
                            p_hbm.at[c, pl.ds(tail0, n - tail0)])
            pltpu.sync_copy(
                hist_sh.at[pl.ds(tail0 * n_cols, (n - tail0) * n_cols)],
                h_hbm.at[c, pl.ds(tail0 * n_cols, (n - tail0) * n_cols)])

    zr = jnp.zeros((_C, d), jnp.float32)
    zh = jnp.zeros((hflat,), jnp.float32)
    pos1 = jnp.ones((_C,), jnp.float32)
    neg1 = jnp.full((_C,), -1.0, jnp.float32)
    return run(xk, src, dst, hidx, zr, zh, pos1, neg1)


def _tc_prepass(x, kernels, bn_gamma, bn_beta):
    n, d = x.shape

    def body(x_ref, k_ref, g_ref, b_ref, o_ref):
        xv = x_ref[...]
        mean = jnp.mean(xv, axis=0)
        xc = xv - mean[None, :]
        var = jnp.mean(xc * xc, axis=0)
        sc = g_ref[0, :] / jnp.sqrt(var + _EPS)
        o_ref[...] = (xc * sc[None, :] + b_ref[0, :][None, :]) * k_ref[0, :]

    return pl.pallas_call(
        body,
        out_shape=jax.ShapeDtypeStruct((n, d), jnp.float32),
    )(x, kernels, bn_gamma, bn_beta)


def _tc_combine(xk, p, h, r, kernels, escal):
    n, d = xk.shape

    def body(x_ref, p_ref, h_ref, r_ref, k_ref, es_ref, o_ref):
        rk = r_ref[...] * k_ref[0, :][None, :]
        pv = p_ref[0] + p_ref[1]
        hv = h_ref[0] + h_ref[1]
        relpart = jnp.dot(hv, rk, preferred_element_type=jnp.float32,
                          precision=lax.Precision.HIGHEST)
        num = pv + relpart + x_ref[...]
        esv = es_ref[...]
        iot = lax.broadcasted_iota(jnp.int32, (n, 1), 0)
        du = jnp.ones((n, 1), jnp.float32)
        for i in range(3):
            ai = esv[2 * i:2 * i + 1, :]
            bi = esv[2 * i + 1:2 * i + 2, :]
            inc = (ai != bi).astype(jnp.float32)
            du = du + inc * ((iot == ai).astype(jnp.float32)
                             + (iot == bi).astype(jnp.float32))
        o_ref[...] = num / du

    return pl.pallas_call(
        body,
        out_shape=jax.ShapeDtypeStruct((n, d), jnp.float32),
    )(xk, p, h, r, kernels, escal)


def kernel(x, edges, rels, r, kernels, bn_gamma, bn_beta):
    n, d = x.shape
    e = edges.shape[1]
    nrel = r.shape[0]
    e0 = edges[0].astype(jnp.int32)
    e1 = edges[1].astype(jnp.int32)
    rl = rels.astype(jnp.int32)
    m = 2 * e
    chunk = _NW * _C * _SB
    mp = -(-m // chunk) * chunk
    gpt = mp // (_NW * _C)
    pad = mp - m
    dump = n + jnp.arange(pad, dtype=jnp.int32) % 128
    src = jnp.concatenate([e0, e1, jnp.zeros((pad,), jnp.int32)])
    dst = jnp.concatenate([e1, e0, dump])
    col = jnp.concatenate([rl, rl, jnp.zeros((pad,), jnp.int32)])
    hidx = dst * nrel + col
    xk = _tc_prepass(x, kernels, bn_gamma.reshape(1, d),
                     bn_beta.reshape(1, d))
    p, hf = _sc_message_pass(xk, src.reshape(-1, _C), dst.reshape(-1, _C),
                             hidx.reshape(-1, _C), nrel, gpt, e // _C,
                             -(-m // _C))
    h = hf.reshape(2, n, nrel)
    escal = jnp.stack([e0[0], e0[2], rl[0], rl[2], e1[0], e1[2],
                       jnp.zeros((), jnp.int32), jnp.zeros((), jnp.int32)])
    return _tc_combine(xk, p, h, r, kernels, escal.reshape(8, 1))


# speedup vs baseline: 3.7657x; 1.0640x over previous
import functools

import jax
import jax.numpy as jnp
from jax import lax
from jax.experimental import pallas as pl
from jax.experimental.pallas import tpu as pltpu
from jax.experimental.pallas import tpu_sc as plsc

_C = 128
_NW = 32
_SB = 8
_EPS = 1e-5


def _sc_message_pass(xk, src, dst, hidx, n_cols, gpt, e_groups, rg):
    n, d = xk.shape
    rows_pad = n
    fl_rows = (n // 16) // 8 * 8
    tail0 = fl_rows * 16
    hflat = rows_pad * n_cols
    nsb = gpt // _SB

    mesh = plsc.VectorSubcoreMesh(core_axis_name="c", subcore_axis_name="s")

    @functools.partial(
        pl.kernel,
        out_type=(
            jax.ShapeDtypeStruct((2, n, d), jnp.float32),
            jax.ShapeDtypeStruct((2, n * n_cols), jnp.float32),
        ),
        mesh=mesh,
        scratch_types=(
            pltpu.VMEM((2, _SB, _C), jnp.int32),
            pltpu.VMEM((2, _SB, _C), jnp.int32),
            pltpu.VMEM((2, _SB, _C), jnp.int32),
            pltpu.VMEM((2, _C, d), jnp.float32),
            pltpu.VMEM((_C,), jnp.float32),
            pltpu.VMEM((_C,), jnp.float32),
            pltpu.VMEM_SHARED((rows_pad, d), jnp.float32),
            pltpu.VMEM_SHARED((hflat,), jnp.float32),
            pltpu.SemaphoreType.DMA((2,)),
            pltpu.SemaphoreType.DMA((2,)),
            pltpu.SemaphoreType.DMA((2,)),
            pltpu.SemaphoreType.DMA,
        ),
    )
    def run(x_hbm, src_hbm, dst_hbm, hidx_hbm, zr_hbm, zh_hbm, pos_hbm,
            neg_hbm, p_hbm, h_hbm,
            src_i, dst_i, hid_i, bufs_v, pos_v, neg_v, acc_sh, hist_sh,
            gsem, ssem, isem, hsem):
        c = lax.axis_index("c")
        s = lax.axis_index("s")
        w = c * 16 + s
        rbase = w * gpt

        pltpu.sync_copy(pos_hbm, pos_v)
        pltpu.sync_copy(neg_hbm, neg_v)

        z0 = bufs_v.at[0]
        pltpu.sync_copy(zr_hbm, z0)
        zb = s * fl_rows
        nchunks = (fl_rows + _C - 1) // _C
        for k in range(nchunks):
            m = min(_C, fl_rows - k * _C)
            pltpu.sync_copy(z0.at[pl.ds(0, m)],
                            acc_sh.at[pl.ds(zb + k * _C, m)])

        @pl.when(s == 15)
        def _zero_tail():
            left = rows_pad - tail0
            off = tail0
            while left > 0:
                mm = min(_C, left)
                pltpu.sync_copy(z0.at[pl.ds(0, mm)],
                                acc_sh.at[pl.ds(off, mm)])
                off += mm
                left -= mm

        @pl.when(s == 0)
        def _zero_hist():
            pltpu.sync_copy(zh_hbm, hist_sh)

        plsc.subcore_barrier()

        pltpu.sync_copy(src_hbm.at[pl.ds(rbase, _SB)], src_i.at[0])
        pltpu.sync_copy(dst_hbm.at[pl.ds(rbase, _SB)], dst_i.at[0])
        pltpu.sync_copy(hidx_hbm.at[pl.ds(rbase, _SB)], hid_i.at[0])
        pltpu.async_copy(x_hbm.at[src_i.at[0, 0]], bufs_v.at[0], gsem.at[0])

        def body(g, carry):
            sb = g // _SB
            pos = g - sb * _SB
            slot = lax.rem(sb, 2)
            b = lax.rem(g, 2)
            gg = rbase + g

            @pl.when(jnp.logical_and(pos == 0,
                                     jnp.logical_and(g + _SB < gpt,
                                                     gg + _SB < rg)))
            def _prefetch():
                nslot = 1 - slot
                nrow = rbase + (sb + 1) * _SB
                pltpu.async_copy(src_hbm.at[pl.ds(nrow, _SB)],
                                 src_i.at[nslot], isem.at[nslot])
                pltpu.async_copy(dst_hbm.at[pl.ds(nrow, _SB)],
                                 dst_i.at[nslot], isem.at[nslot])
                pltpu.async_copy(hidx_hbm.at[pl.ds(nrow, _SB)],
                                 hid_i.at[nslot], isem.at[nslot])

            @pl.when(gg < rg)
            def _process():
                pltpu.make_async_copy(x_hbm.at[src_i.at[slot, pos]],
                                      bufs_v.at[b], gsem.at[b]).wait()
                pltpu.async_copy(bufs_v.at[b], acc_sh.at[dst_i.at[slot, pos]],
                                 ssem.at[b], add=True)

                @pl.when(gg < e_groups)
                def _pos_hist():
                    pltpu.async_copy(pos_v, hist_sh.at[hid_i.at[slot, pos]],
                                     hsem, add=True)

                @pl.when(gg >= e_groups)
                def _neg_hist():
                    pltpu.async_copy(neg_v, hist_sh.at[hid_i.at[slot, pos]],
                                     hsem, add=True)

            @pl.when(jnp.logical_and(g + 1 < gpt, gg + 1 < rg))
            def _next_gather():
                bn = 1 - b

                @pl.when(g > 0)
                def _wait_prev_scatter():
                    pltpu.make_async_copy(zr_hbm, bufs_v.at[bn],
                                          ssem.at[bn]).wait()

                gn = g + 1
                sbn = gn // _SB
                posn = gn - sbn * _SB
                slotn = lax.rem(sbn, 2)

                @pl.when(posn == 0)
                def _wait_idx():
                    pltpu.make_async_copy(src_hbm.at[pl.ds(0, _SB)],
                                          src_i.at[slotn],
                                          isem.at[slotn]).wait()
                    pltpu.make_async_copy(dst_hbm.at[pl.ds(0, _SB)],
                                          dst_i.at[slotn],
                                          isem.at[slotn]).wait()
                    pltpu.make_async_copy(hidx_hbm.at[pl.ds(0, _SB)],
                                          hid_i.at[slotn],
                                          isem.at[slotn]).wait()

                pltpu.async_copy(x_hbm.at[src_i.at[slotn, posn]],
                                 bufs_v.at[bn], gsem.at[bn])

            return carry

        lax.fori_loop(0, gpt, body, 0)
        for b in range(2):
            pltpu.make_async_copy(zr_hbm, bufs_v.at[b], ssem.at[b]).wait()
        nreal = jnp.clip(rg - rbase, 0, gpt)

        def _drain(i, carry):
            pltpu.make_async_copy(pos_hbm, pos_v, hsem).wait()
            return carry

        lax.fori_loop(0, nreal, _drain, 0)
        plsc.subcore_barrier()

        fb = s * fl_rows
        pltpu.sync_copy(acc_sh.at[pl.ds(fb, fl_rows)],
                        p_hbm.at[c, pl.ds(fb, fl_rows)])
        pltpu.sync_copy(hist_sh.at[pl.ds(fb * n_cols, fl_rows * n_cols)],
                        h_hbm.at[c, pl.ds(fb * n_cols, fl_rows * n_cols)])

        @pl.when(s == 15)
        def _flush_tail():
            pltpu.sync_copy(acc_sh.at[pl.ds(tail0, n - tail0)],
                            p_hbm.at[c, pl.ds(tail0, n - tail0)])
            pltpu.sync_copy(
                hist_sh.at[pl.ds(tail0 * n_cols, (n - tail0) * n_cols)],
                h_hbm.at[c, pl.ds(tail0 * n_cols, (n - tail0) * n_cols)])

    zr = jnp.zeros((_C, d), jnp.float32)
    zh = jnp.zeros((hflat,), jnp.float32)
    pos1 = jnp.ones((_C,), jnp.float32)
    neg1 = jnp.full((_C,), -1.0, jnp.float32)
    return run(xk, src, dst, hidx, zr, zh, pos1, neg1)


def _tc_prepass(x, kernels, bn_gamma, bn_beta):
    n, d = x.shape

    def body(x_ref, k_ref, g_ref, b_ref, o_ref):
        xv = x_ref[...]
        mean = jnp.mean(xv, axis=0)
        xc = xv - mean[None, :]
        var = jnp.mean(xc * xc, axis=0)
        sc = g_ref[0, :] / jnp.sqrt(var + _EPS)
        o_ref[...] = (xc * sc[None, :] + b_ref[0, :][None, :]) * k_ref[0, :]

    return pl.pallas_call(
        body,
        out_shape=jax.ShapeDtypeStruct((n, d), jnp.float32),
    )(x, kernels, bn_gamma, bn_beta)


def _tc_combine(xk, p, h, r, kernels, escal):
    n, d = xk.shape

    def body(x_ref, p_ref, h_ref, r_ref, k_ref, es_ref, o_ref):
        rk = r_ref[...] * k_ref[0, :][None, :]
        pv = p_ref[0] + p_ref[1]
        hv = h_ref[0] + h_ref[1]
        relpart = jnp.dot(hv, rk, preferred_element_type=jnp.float32,
                          precision=lax.Precision.HIGHEST)
        num = pv + relpart + x_ref[...]
        esv = es_ref[...]
        iot = lax.broadcasted_iota(jnp.int32, (n, 1), 0)
        du = jnp.ones((n, 1), jnp.float32)
        for i in range(3):
            ai = esv[2 * i:2 * i + 1, :]
            bi = esv[2 * i + 1:2 * i + 2, :]
            inc = (ai != bi).astype(jnp.float32)
            du = du + inc * ((iot == ai).astype(jnp.float32)
                             + (iot == bi).astype(jnp.float32))
        o_ref[...] = num / du

    return pl.pallas_call(
        body,
        out_shape=jax.ShapeDtypeStruct((n, d), jnp.float32),
    )(xk, p, h, r, kernels, escal)


def kernel(x, edges, rels, r, kernels, bn_gamma, bn_beta):
    n, d = x.shape
    e = edges.shape[1]
    nrel = r.shape[0]
    e0 = edges[0].astype(jnp.int32)
    e1 = edges[1].astype(jnp.int32)
    rl = rels.astype(jnp.int32)
    m = 2 * e
    chunk = _NW * _C * _SB
    mp = -(-m // chunk) * chunk
    gpt = mp // (_NW * _C)
    pad = mp - m
    dump = n + jnp.arange(pad, dtype=jnp.int32) % 128
    src = jnp.concatenate([e0, e1, jnp.zeros((pad,), jnp.int32)])
    dst = jnp.concatenate([e1, e0, dump])
    col = jnp.concatenate([rl, rl, jnp.zeros((pad,), jnp.int32)])
    hidx = dst * nrel + col
    xk = _tc_prepass(x, kernels, bn_gamma.reshape(1, d),
                     bn_beta.reshape(1, d))
    p, hf = _sc_message_pass(xk, src.reshape(-1, _C), dst.reshape(-1, _C),
                             hidx.reshape(-1, _C), nrel, gpt, e // _C,
                             -(-m // _C))
    h = hf.reshape(2, n, nrel)
    escal = jnp.stack([e0[0], e0[2], rl[0], rl[2], e1[0], e1[2],
                       jnp.zeros((), jnp.int32), jnp.zeros((), jnp.int32)])
    return _tc_combine(xk, p, h, r, kernels, escal.reshape(8, 1))
